# baseline (speedup 1.0000x reference)
import functools

import jax
import jax.numpy as jnp
from jax import lax
from jax.experimental import pallas as pl
from jax.experimental.pallas import tpu as pltpu
from jax.experimental.pallas import tpu_sc as plsc

_L = 16


def _make_sc_kernel(CB, V2, D, n_params):
    info = plsc.get_sparse_core_info()
    NC, NS = info.num_cores, info.num_subcores
    NW = NC * NS
    npw = CB // NW
    nchunk = npw // 128
    nblk = npw // _L
    assert npw % 128 == 0 and CB % NW == 0

    mesh = plsc.VectorSubcoreMesh(core_axis_name="c", subcore_axis_name="s")

    @functools.partial(
        pl.kernel,
        mesh=mesh,
        out_type=jax.ShapeDtypeStruct((CB,), jnp.float32),
        scratch_types=[
            pltpu.VMEM((nchunk, 128), jnp.int32),
            pltpu.VMEM((nchunk, 128), jnp.int32),
            pltpu.VMEM((npw, D), jnp.float32),
            pltpu.VMEM((npw, D), jnp.float32),
            pltpu.VMEM((npw,), jnp.float32),
            pltpu.VMEM((n_params,), jnp.float32),
            pltpu.SemaphoreType.DMA,
        ],
        compiler_params=pltpu.CompilerParams(
            needs_layout_passes=False, use_tc_tiling_on_sc=False),
    )
    def sc_kernel(uidx_hbm, iidx_hbm, utab_hbm, itab_hbm, params_hbm, out_hbm,
                  uidx_v, iidx_v, urows_v, irows_v, out_v, params_v,
                  sem):
        wid = lax.axis_index("s") * NC + lax.axis_index("c")

        pltpu.sync_copy(uidx_hbm.at[pl.ds(wid * nchunk, nchunk)], uidx_v)
        pltpu.sync_copy(iidx_hbm.at[pl.ds(wid * nchunk, nchunk)], iidx_v)

        copies = []
        for j in range(nchunk):
            copies.append(pltpu.async_copy(
                utab_hbm.at[uidx_v.at[j]], urows_v.at[pl.ds(j * 128, 128)], sem))
        for j in range(nchunk):
            copies.append(pltpu.async_copy(
                itab_hbm.at[iidx_v.at[j]], irows_v.at[pl.ds(j * 128, 128)], sem))

        pltpu.sync_copy(params_hbm, params_v)
        iota = lax.iota(jnp.int32, _L)
        mm_off, mel_off, wf_off = 0, D * 2 * _L, D * 2 * _L + 2 * _L * D
        wfi_vec = params_v[pl.ds(wf_off + D, _L)]
        t0 = jnp.zeros((_L,), jnp.float32)
        t1 = jnp.zeros((_L,), jnp.float32)
        for dd in range(D):
            t0 = t0 + plsc.load_gather(params_v, [mel_off + iota * D + dd]) * wfi_vec[dd]
            t1 = t1 + plsc.load_gather(params_v, [mel_off + (_L + iota) * D + dd]) * wfi_vec[dd]
        t_s = [t0[m] for m in range(_L)] + [t1[m] for m in range(_L)]
        wi = jnp.zeros((_L,), jnp.float32)
        for m in range(2 * _L):
            wi = wi + plsc.load_gather(params_v, [mm_off + iota * (2 * _L) + m]) * t_s[m]

        wu_vec = params_v[pl.ds(wf_off, _L)]
        wu_s = [wu_vec[d] for d in range(D)]
        wi_s = [wi[d] for d in range(D)]
        b_s = params_v[pl.ds(wf_off + 2 * D, _L)][0]

        for cp in copies:
            cp.wait()

        def blk(k, carry):
            r = k * _L + iota
            acc = jnp.zeros((_L,), jnp.float32) + b_s
            for d in range(D):
                dv = jnp.full((_L,), d, jnp.int32)
                acc = acc + plsc.load_gather(urows_v, [r, dv]) * wu_s[d]
                acc = acc + plsc.load_gather(irows_v, [r, dv]) * wi_s[d]
            out_v[pl.ds(k * _L, _L)] = 1.0 / (1.0 + jnp.exp(-acc))
            return carry
        lax.fori_loop(0, nblk, blk, 0)

        pltpu.sync_copy(out_v, out_hbm.at[pl.ds(wid * npw, npw)])

    return sc_kernel


def kernel(inputs, user_tables, item_tables, meta_memory, meta_emb_layer,
           w_final, b_final):
    C, B, _ = inputs.shape
    _, V, D = user_tables.shape
    M = meta_memory.shape[1]
    CB = C * B

    ids = inputs.astype(jnp.int32)
    offs = (jnp.arange(C, dtype=jnp.int32) * V)[:, None]
    uidx = (ids[:, :, 0] + offs).reshape(CB // 128, 128)
    iidx = (ids[:, :, 1] + offs).reshape(CB // 128, 128)
    utab = user_tables.reshape(C * V, D)
    itab = item_tables.reshape(C * V, D)
    raw = [meta_memory.reshape(-1), meta_emb_layer.reshape(-1),
           w_final.reshape(-1).astype(jnp.float32),
           b_final.reshape(-1).astype(jnp.float32)]
    n_raw = D * M + M * D + 2 * D + 1
    n_params = ((n_raw + 15) // 16) * 16 + 16
    raw.append(jnp.zeros((n_params - n_raw,), jnp.float32))
    params = jnp.concatenate(raw)

    sc = _make_sc_kernel(CB, C * V, D, n_params)
    out = sc(uidx, iidx, utab, itab, params)
    out = out.reshape(C, B)
    return tuple(out[i] for i in range(C))


# speedup vs baseline: 10.9792x; 10.9792x over previous
import functools

import jax
import jax.numpy as jnp
from jax import lax
from jax.experimental import pallas as pl
from jax.experimental.pallas import tpu as pltpu
from jax.experimental.pallas import tpu_sc as plsc

_L = 16
_VB = 65536


def _make_tc_project(C, D, V, nv):
    vb8 = _VB // 8

    def body(u_ref, i_ref, mm_ref, mel_ref, wf_ref, su_ref, si_ref):
        wf = wf_ref[...]
        wu = wf[:D, 0]
        wi = (mm_ref[...] @ (mel_ref[...] @ wf[D:]))[:, 0]
        u = u_ref[0]
        i = i_ref[0]
        su_ref[...] = jnp.sum(u * wu[:, None], axis=0).reshape(8, vb8)
        si_ref[...] = jnp.sum(i * wi[:, None], axis=0).reshape(8, vb8)

    grid = (C, nv)
    return pl.pallas_call(
        body,
        grid=grid,
        in_specs=[
            pl.BlockSpec((1, D, _VB), lambda c, j: (c, 0, j)),
            pl.BlockSpec((1, D, _VB), lambda c, j: (c, 0, j)),
            pl.BlockSpec((D, 2 * D), lambda c, j: (0, 0)),
            pl.BlockSpec((2 * D, D), lambda c, j: (0, 0)),
            pl.BlockSpec((2 * D, 1), lambda c, j: (0, 0)),
        ],
        out_specs=[
            pl.BlockSpec((8, vb8), lambda c, j: (c * nv + j, 0)),
            pl.BlockSpec((8, vb8), lambda c, j: (c * nv + j, 0)),
        ],
        out_shape=[
            jax.ShapeDtypeStruct((C * nv * 8, vb8), jnp.float32),
            jax.ShapeDtypeStruct((C * nv * 8, vb8), jnp.float32),
        ],
    )


def _make_sc_gather(CB):
    info = plsc.get_sparse_core_info()
    NC, NS = info.num_cores, info.num_subcores
    NW = NC * NS
    npw = CB // NW
    nchunk = npw // 128
    nblk = npw // _L
    assert CB % NW == 0 and npw % 128 == 0

    mesh = plsc.VectorSubcoreMesh(core_axis_name="c", subcore_axis_name="s")

    @functools.partial(
        pl.kernel,
        mesh=mesh,
        out_type=jax.ShapeDtypeStruct((CB,), jnp.float32),
        scratch_types=[
            pltpu.VMEM((nchunk, 128), jnp.int32),
            pltpu.VMEM((nchunk, 128), jnp.int32),
            pltpu.VMEM((npw,), jnp.float32),
            pltpu.VMEM((npw,), jnp.float32),
            pltpu.VMEM((npw,), jnp.float32),
            pltpu.VMEM((_L,), jnp.float32),
            pltpu.SemaphoreType.DMA,
        ],
        compiler_params=pltpu.CompilerParams(
            needs_layout_passes=False, use_tc_tiling_on_sc=False),
    )
    def sc_kernel(uidx_hbm, iidx_hbm, su_hbm, si_hbm, bias_hbm, out_hbm,
                  uidx_v, iidx_v, sgu_v, sgi_v, out_v, bias_v, sem):
        wid = lax.axis_index("s") * NC + lax.axis_index("c")

        pltpu.sync_copy(uidx_hbm.at[pl.ds(wid * nchunk, nchunk)], uidx_v)
        pltpu.sync_copy(iidx_hbm.at[pl.ds(wid * nchunk, nchunk)], iidx_v)

        copies = []
        for j in range(nchunk):
            copies.append(pltpu.async_copy(
                su_hbm.at[uidx_v.at[j]], sgu_v.at[pl.ds(j * 128, 128)], sem))
        for j in range(nchunk):
            copies.append(pltpu.async_copy(
                si_hbm.at[iidx_v.at[j]], sgi_v.at[pl.ds(j * 128, 128)], sem))

        pltpu.sync_copy(bias_hbm, bias_v)
        b_s = bias_v[pl.ds(0, _L)][0]

        for cp in copies:
            cp.wait()

        def blk(k, carry):
            x = sgu_v[pl.ds(k * _L, _L)] + sgi_v[pl.ds(k * _L, _L)] + b_s
            out_v[pl.ds(k * _L, _L)] = 1.0 / (1.0 + jnp.exp(-x))
            return carry
        lax.fori_loop(0, nblk, blk, 0)

        pltpu.sync_copy(out_v, out_hbm.at[pl.ds(wid * npw, npw)])

    return sc_kernel


def kernel(inputs, user_tables, item_tables, meta_memory, meta_emb_layer,
           w_final, b_final):
    C, B, _ = inputs.shape
    _, V, D = user_tables.shape
    CB = C * B
    nv = (V + _VB - 1) // _VB
    vpad = nv * _VB

    utab_t = user_tables.transpose(0, 2, 1)
    itab_t = item_tables.transpose(0, 2, 1)
    ids = inputs.astype(jnp.int32)
    offs = (jnp.arange(C, dtype=jnp.int32) * vpad)[:, None]
    uidx = (ids[:, :, 0] + offs).reshape(CB // 128, 128)
    iidx = (ids[:, :, 1] + offs).reshape(CB // 128, 128)

    tc = _make_tc_project(C, D, V, nv)
    su2, si2 = tc(utab_t, itab_t, meta_memory, meta_emb_layer,
                  w_final.astype(jnp.float32))
    su = su2.reshape(C * vpad)
    si = si2.reshape(C * vpad)

    bias = jnp.broadcast_to(b_final.astype(jnp.float32), (_L,))

    sc = _make_sc_gather(CB)
    out = sc(uidx, iidx, su, si, bias)
    out = out.reshape(C, B)
    return tuple(out[i] for i in range(C))


# speedup vs baseline: 11.4613x; 1.0439x over previous
import functools

import jax
import jax.numpy as jnp
from jax import lax
from jax.experimental import pallas as pl
from jax.experimental.pallas import tpu as pltpu
from jax.experimental.pallas import tpu_sc as plsc

_L = 16
_VB = 131072


def _make_tc_project(C, D, V, nv):
    vb8 = _VB // 8

    def body(u_ref, i_ref, mm_ref, mel_ref, wf_ref, su_ref, si_ref):
        wf = wf_ref[...]
        wu = wf[:D, 0]
        wi = (mm_ref[...] @ (mel_ref[...] @ wf[D:]))[:, 0]
        u = u_ref[0]
        i = i_ref[0]
        su_ref[...] = jnp.sum(u * wu[:, None], axis=0).reshape(8, vb8)
        si_ref[...] = jnp.sum(i * wi[:, None], axis=0).reshape(8, vb8)

    grid = (C, nv)
    return pl.pallas_call(
        body,
        grid=grid,
        in_specs=[
            pl.BlockSpec((1, D, _VB), lambda c, j: (c, 0, j)),
            pl.BlockSpec((1, D, _VB), lambda c, j: (c, 0, j)),
            pl.BlockSpec((D, 2 * D), lambda c, j: (0, 0)),
            pl.BlockSpec((2 * D, D), lambda c, j: (0, 0)),
            pl.BlockSpec((2 * D, 1), lambda c, j: (0, 0)),
        ],
        out_specs=[
            pl.BlockSpec((8, vb8), lambda c, j: (c * nv + j, 0)),
            pl.BlockSpec((8, vb8), lambda c, j: (c * nv + j, 0)),
        ],
        out_shape=[
            jax.ShapeDtypeStruct((C * nv * 8, vb8), jnp.float32),
            jax.ShapeDtypeStruct((C * nv * 8, vb8), jnp.float32),
        ],
    )


def _make_sc_gather(CB):
    info = plsc.get_sparse_core_info()
    NC, NS = info.num_cores, info.num_subcores
    NW = NC * NS
    npw = CB // NW
    nchunk = npw // 128
    nblk = npw // _L
    B = CB // 2
    wpc = NW // 2
    assert CB % NW == 0 and npw % 128 == 0 and B % npw == 0

    mesh = plsc.VectorSubcoreMesh(core_axis_name="c", subcore_axis_name="s")

    @functools.partial(
        pl.kernel,
        mesh=mesh,
        out_type=[jax.ShapeDtypeStruct((B,), jnp.float32),
                  jax.ShapeDtypeStruct((B,), jnp.float32)],
        scratch_types=[
            pltpu.VMEM((nchunk, 128), jnp.int32),
            pltpu.VMEM((nchunk, 128), jnp.int32),
            pltpu.VMEM((npw,), jnp.float32),
            pltpu.VMEM((npw,), jnp.float32),
            pltpu.VMEM((npw,), jnp.float32),
            pltpu.VMEM((_L,), jnp.float32),
            pltpu.SemaphoreType.DMA,
        ],
        compiler_params=pltpu.CompilerParams(
            needs_layout_passes=False, use_tc_tiling_on_sc=False),
    )
    def sc_kernel(uidx_hbm, iidx_hbm, su_hbm, si_hbm, bias_hbm,
                  out0_hbm, out1_hbm,
                  uidx_v, iidx_v, sgu_v, sgi_v, out_v, bias_v, sem):
        wid = lax.axis_index("s") * NC + lax.axis_index("c")

        pltpu.sync_copy(uidx_hbm.at[pl.ds(wid * nchunk, nchunk)], uidx_v)
        pltpu.sync_copy(iidx_hbm.at[pl.ds(wid * nchunk, nchunk)], iidx_v)

        copies = []
        for j in range(nchunk):
            copies.append(pltpu.async_copy(
                su_hbm.at[uidx_v.at[j]], sgu_v.at[pl.ds(j * 128, 128)], sem))
        for j in range(nchunk):
            copies.append(pltpu.async_copy(
                si_hbm.at[iidx_v.at[j]], sgi_v.at[pl.ds(j * 128, 128)], sem))

        pltpu.sync_copy(bias_hbm, bias_v)
        b_s = bias_v[pl.ds(0, _L)][0]

        for cp in copies:
            cp.wait()

        def blk(k, carry):
            x = sgu_v[pl.ds(k * _L, _L)] + sgi_v[pl.ds(k * _L, _L)] + b_s
            out_v[pl.ds(k * _L, _L)] = 1.0 / (1.0 + jnp.exp(-x))
            return carry
        lax.fori_loop(0, nblk, blk, 0)

        @pl.when(wid < wpc)
        def _():
            pltpu.sync_copy(out_v, out0_hbm.at[pl.ds(wid * npw, npw)])

        @pl.when(wid >= wpc)
        def _():
            pltpu.sync_copy(out_v, out1_hbm.at[pl.ds((wid - wpc) * npw, npw)])

    return sc_kernel


def kernel(inputs, user_tables, item_tables, meta_memory, meta_emb_layer,
           w_final, b_final):
    C, B, _ = inputs.shape
    _, V, D = user_tables.shape
    CB = C * B
    nv = (V + _VB - 1) // _VB
    vpad = nv * _VB

    utab_t = user_tables.transpose(0, 2, 1)
    itab_t = item_tables.transpose(0, 2, 1)
    ids = inputs.astype(jnp.int32)
    offs = (jnp.arange(C, dtype=jnp.int32) * vpad)[:, None]
    uidx = (ids[:, :, 0] + offs).reshape(CB // 128, 128)
    iidx = (ids[:, :, 1] + offs).reshape(CB // 128, 128)

    tc = _make_tc_project(C, D, V, nv)
    su2, si2 = tc(utab_t, itab_t, meta_memory, meta_emb_layer,
                  w_final.astype(jnp.float32))
    su = su2.reshape(C * vpad)
    si = si2.reshape(C * vpad)

    bias = jnp.broadcast_to(b_final.astype(jnp.float32), (_L,))

    sc = _make_sc_gather(CB)
    out0, out1 = sc(uidx, iidx, su, si, bias)
    return (out0, out1)


# speedup vs baseline: 11.8414x; 1.0332x over previous
import functools

import jax
import jax.numpy as jnp
from jax import lax
from jax.experimental import pallas as pl
from jax.experimental.pallas import tpu as pltpu
from jax.experimental.pallas import tpu_sc as plsc

_L = 16
_VB = 131072


def _make_tc_project(C, D, V, nv):
    vb8 = _VB // 8

    def body(u_ref, i_ref, mm_ref, mel_ref, wf_ref, s_ref):
        wf = wf_ref[...]
        wu = wf[:D, 0]
        wi = (mm_ref[...] @ (mel_ref[...] @ wf[D:]))[:, 0]
        u = u_ref[0]
        i = i_ref[0]
        s_ref[0] = jnp.sum(u * wu[:, None], axis=0).reshape(8, vb8)
        s_ref[1] = jnp.sum(i * wi[:, None], axis=0).reshape(8, vb8)

    grid = (C, nv)
    return pl.pallas_call(
        body,
        grid=grid,
        in_specs=[
            pl.BlockSpec((1, D, _VB), lambda c, j: (c, 0, j)),
            pl.BlockSpec((1, D, _VB), lambda c, j: (c, 0, j)),
            pl.BlockSpec((D, 2 * D), lambda c, j: (0, 0)),
            pl.BlockSpec((2 * D, D), lambda c, j: (0, 0)),
            pl.BlockSpec((2 * D, 1), lambda c, j: (0, 0)),
        ],
        out_specs=pl.BlockSpec((2, 8, vb8), lambda c, j: (0, c * nv + j, 0)),
        out_shape=jax.ShapeDtypeStruct((2, C * nv * 8, vb8), jnp.float32),
    )


def _make_sc_gather(CB):
    info = plsc.get_sparse_core_info()
    NC, NS = info.num_cores, info.num_subcores
    NW = NC * NS
    npw = CB // NW
    nchunk = npw // 128
    nblk = npw // _L
    B = CB // 2
    wpc = NW // 2
    assert CB % NW == 0 and npw % 128 == 0 and B % npw == 0

    mesh = plsc.VectorSubcoreMesh(core_axis_name="c", subcore_axis_name="s")

    @functools.partial(
        pl.kernel,
        mesh=mesh,
        out_type=[jax.ShapeDtypeStruct((B,), jnp.float32),
                  jax.ShapeDtypeStruct((B,), jnp.float32)],
        scratch_types=[
            pltpu.VMEM((nchunk, 128), jnp.int32),
            pltpu.VMEM((nchunk, 128), jnp.int32),
            pltpu.VMEM((npw,), jnp.float32),
            pltpu.VMEM((npw,), jnp.float32),
            pltpu.VMEM((npw,), jnp.float32),
            pltpu.VMEM((_L,), jnp.float32),
            pltpu.SemaphoreType.DMA,
        ],
        compiler_params=pltpu.CompilerParams(
            needs_layout_passes=False, use_tc_tiling_on_sc=False),
    )
    def sc_kernel(uidx_hbm, iidx_hbm, s_hbm, bias_hbm, out0_hbm, out1_hbm,
                  uidx_v, iidx_v, sgu_v, sgi_v, out_v, bias_v, sem):
        wid = lax.axis_index("s") * NC + lax.axis_index("c")

        pltpu.sync_copy(uidx_hbm.at[pl.ds(wid * nchunk, nchunk)], uidx_v)
        pltpu.sync_copy(iidx_hbm.at[pl.ds(wid * nchunk, nchunk)], iidx_v)

        copies = []
        for j in range(nchunk):
            copies.append(pltpu.async_copy(
                s_hbm.at[uidx_v.at[j]], sgu_v.at[pl.ds(j * 128, 128)], sem))
        for j in range(nchunk):
            copies.append(pltpu.async_copy(
                s_hbm.at[iidx_v.at[j]], sgi_v.at[pl.ds(j * 128, 128)], sem))

        pltpu.sync_copy(bias_hbm, bias_v)
        b_s = bias_v[pl.ds(0, _L)][0]

        for cp in copies:
            cp.wait()

        def blk(k, carry):
            x = sgu_v[pl.ds(k * _L, _L)] + sgi_v[pl.ds(k * _L, _L)] + b_s
            out_v[pl.ds(k * _L, _L)] = 1.0 / (1.0 + jnp.exp(-x))
            return carry
        lax.fori_loop(0, nblk, blk, 0)

        @pl.when(wid < wpc)
        def _():
            pltpu.sync_copy(out_v, out0_hbm.at[pl.ds(wid * npw, npw)])

        @pl.when(wid >= wpc)
        def _():
            pltpu.sync_copy(out_v, out1_hbm.at[pl.ds((wid - wpc) * npw, npw)])

    return sc_kernel


def kernel(inputs, user_tables, item_tables, meta_memory, meta_emb_layer,
           w_final, b_final):
    C, B, _ = inputs.shape
    _, V, D = user_tables.shape
    CB = C * B
    nv = (V + _VB - 1) // _VB
    vpad = nv * _VB

    utab_t = user_tables.transpose(0, 2, 1)
    itab_t = item_tables.transpose(0, 2, 1)
    ids = inputs.astype(jnp.int32)
    offs = (jnp.arange(C, dtype=jnp.int32) * vpad)[:, None]
    uidx = (ids[:, :, 0] + offs).reshape(CB // 128, 128)
    iidx = (ids[:, :, 1] + offs + C * vpad).reshape(CB // 128, 128)

    tc = _make_tc_project(C, D, V, nv)
    s2 = tc(utab_t, itab_t, meta_memory, meta_emb_layer,
            w_final.astype(jnp.float32))
    s = s2.reshape(2 * C * vpad)

    bias = jnp.broadcast_to(b_final.astype(jnp.float32), (_L,))

    sc = _make_sc_gather(CB)
    out0, out1 = sc(uidx, iidx, s, bias)
    return (out0, out1)


# speedup vs baseline: 12.0829x; 1.0204x over previous
import functools

import jax
import jax.numpy as jnp
from jax import lax
from jax.experimental import pallas as pl
from jax.experimental.pallas import tpu as pltpu
from jax.experimental.pallas import tpu_sc as plsc

_L = 16
_VB = 147456


def _make_tc_project(C, D, V, nv):
    vb8 = _VB // 8

    def body(u_ref, i_ref, mm_ref, mel_ref, wf_ref, s_ref):
        wf = wf_ref[...]
        wu = wf[:D, 0]
        wi = (mm_ref[...] @ (mel_ref[...] @ wf[D:]))[:, 0]
        u = u_ref[0]
        i = i_ref[0]
        s_ref[0] = jnp.sum(u * wu[:, None], axis=0).reshape(8, vb8)
        s_ref[1] = jnp.sum(i * wi[:, None], axis=0).reshape(8, vb8)

    grid = (C, nv)
    return pl.pallas_call(
        body,
        grid=grid,
        in_specs=[
            pl.BlockSpec((1, D, _VB), lambda c, j: (c, 0, j)),
            pl.BlockSpec((1, D, _VB), lambda c, j: (c, 0, j)),
            pl.BlockSpec((D, 2 * D), lambda c, j: (0, 0)),
            pl.BlockSpec((2 * D, D), lambda c, j: (0, 0)),
            pl.BlockSpec((2 * D, 1), lambda c, j: (0, 0)),
        ],
        out_specs=pl.BlockSpec((2, 8, vb8), lambda c, j: (0, c * nv + j, 0)),
        out_shape=jax.ShapeDtypeStruct((2, C * nv * 8, vb8), jnp.float32),
        compiler_params=pltpu.CompilerParams(vmem_limit_bytes=50 * 1024 * 1024),
    )


def _make_sc_gather(CB):
    info = plsc.get_sparse_core_info()
    NC, NS = info.num_cores, info.num_subcores
    NW = NC * NS
    npw = CB // NW
    nchunk = npw // 128
    nblk = npw // _L
    B = CB // 2
    wpc = NW // 2
    assert CB % NW == 0 and npw % 128 == 0 and B % npw == 0

    mesh = plsc.VectorSubcoreMesh(core_axis_name="c", subcore_axis_name="s")

    @functools.partial(
        pl.kernel,
        mesh=mesh,
        out_type=[jax.ShapeDtypeStruct((B,), jnp.float32),
                  jax.ShapeDtypeStruct((B,), jnp.float32)],
        scratch_types=[
            pltpu.VMEM((nchunk, 128), jnp.int32),
            pltpu.VMEM((nchunk, 128), jnp.int32),
            pltpu.VMEM((npw,), jnp.float32),
            pltpu.VMEM((npw,), jnp.float32),
            pltpu.VMEM((npw,), jnp.float32),
            pltpu.VMEM((_L,), jnp.float32),
            pltpu.SemaphoreType.DMA,
        ],
        compiler_params=pltpu.CompilerParams(
            needs_layout_passes=False, use_tc_tiling_on_sc=False),
    )
    def sc_kernel(uidx_hbm, iidx_hbm, s_hbm, bias_hbm, out0_hbm, out1_hbm,
                  uidx_v, iidx_v, sgu_v, sgi_v, out_v, bias_v, sem):
        wid = lax.axis_index("s") * NC + lax.axis_index("c")

        pltpu.sync_copy(uidx_hbm.at[pl.ds(wid * nchunk, nchunk)], uidx_v)
        pltpu.sync_copy(iidx_hbm.at[pl.ds(wid * nchunk, nchunk)], iidx_v)

        copies = []
        for j in range(nchunk):
            copies.append(pltpu.async_copy(
                s_hbm.at[uidx_v.at[j]], sgu_v.at[pl.ds(j * 128, 128)], sem))
        for j in range(nchunk):
            copies.append(pltpu.async_copy(
                s_hbm.at[iidx_v.at[j]], sgi_v.at[pl.ds(j * 128, 128)], sem))

        pltpu.sync_copy(bias_hbm, bias_v)
        b_s = bias_v[pl.ds(0, _L)][0]

        for cp in copies:
            cp.wait()

        def blk(k, carry):
            x = sgu_v[pl.ds(k * _L, _L)] + sgi_v[pl.ds(k * _L, _L)] + b_s
            out_v[pl.ds(k * _L, _L)] = 1.0 / (1.0 + jnp.exp(-x))
            return carry
        lax.fori_loop(0, nblk, blk, 0)

        @pl.when(wid < wpc)
        def _():
            pltpu.sync_copy(out_v, out0_hbm.at[pl.ds(wid * npw, npw)])

        @pl.when(wid >= wpc)
        def _():
            pltpu.sync_copy(out_v, out1_hbm.at[pl.ds((wid - wpc) * npw, npw)])

    return sc_kernel


def kernel(inputs, user_tables, item_tables, meta_memory, meta_emb_layer,
           w_final, b_final):
    C, B, _ = inputs.shape
    _, V, D = user_tables.shape
    CB = C * B
    nv = (V + _VB - 1) // _VB
    vpad = nv * _VB

    utab_t = user_tables.transpose(0, 2, 1)
    itab_t = item_tables.transpose(0, 2, 1)
    ids = inputs.astype(jnp.int32)
    offs = (jnp.arange(C, dtype=jnp.int32) * vpad)[:, None]
    uidx = (ids[:, :, 0] + offs).reshape(CB // 128, 128)
    iidx = (ids[:, :, 1] + offs + C * vpad).reshape(CB // 128, 128)

    tc = _make_tc_project(C, D, V, nv)
    s2 = tc(utab_t, itab_t, meta_memory, meta_emb_layer,
            w_final.astype(jnp.float32))
    s = s2.reshape(2 * C * vpad)

    bias = jnp.broadcast_to(b_final.astype(jnp.float32), (_L,))

    sc = _make_sc_gather(CB)
    out0, out1 = sc(uidx, iidx, s, bias)
    return (out0, out1)
